# baseline (speedup 1.0000x reference)
import functools

import jax
import jax.numpy as jnp
import numpy as np
from jax import lax
from jax.experimental import pallas as pl
from jax.experimental.pallas import tpu as pltpu
from jax.experimental.pallas import tpu_sc as plsc

N_SAMPLES = 128
MIN_D = 0.1
MAX_D = 6.0
DZ = (MAX_D - MIN_D) / (N_SAMPLES - 1)
BLOCK_R = 4096
SC_CHUNK = 256
SC_UNROLL = 8


def _threefry2x32_np(k0, k1, x0, x1):
    def rotl(x, d):
        return (x << np.uint32(d)) | (x >> np.uint32(32 - d))

    rots = ((13, 15, 26, 6), (17, 29, 16, 24))
    ks = (k0, k1, k0 ^ k1 ^ np.uint32(0x1BD11BDA))
    x0 = x0 + ks[0]
    x1 = x1 + ks[1]
    for i in range(5):
        for r in rots[i % 2]:
            x0 = x0 + x1
            x1 = rotl(x1, r)
            x1 = x0 ^ x1
        x0 = x0 + ks[(i + 1) % 3]
        x1 = x1 + ks[(i + 2) % 3] + np.uint32(i + 1)
    return x0, x1


@functools.cache
def _u_const(n_rays: int):
    size = n_rays * N_SAMPLES
    with np.errstate(over="ignore"):
        b0, b1 = _threefry2x32_np(
            np.uint32(0), np.uint32(42),
            np.zeros(size, np.uint32), np.arange(size, dtype=np.uint32),
        )
    bits = b0 ^ b1
    u = ((bits >> np.uint32(9)) | np.uint32(0x3F800000)).view(np.float32)
    u = np.maximum(np.float32(0.0), u - np.float32(1.0))
    return u.reshape(n_rays, N_SAMPLES)


@functools.cache
def _bcast_mat():
    e = np.zeros((3, 3 * N_SAMPLES), np.float32)
    for k in range(3):
        e[k, k * N_SAMPLES:(k + 1) * N_SAMPLES] = 1.0
    return jnp.asarray(e)


def _sc_body(dens_hbm, u_hbm, z_hbm, dens_v, u_v, cdf_v, z_v):
    n = N_SAMPLES
    c_rays = SC_CHUNK
    info = plsc.get_sparse_core_info()
    nw = info.num_cores * info.num_subcores
    wid = lax.axis_index("s") * info.num_cores + lax.axis_index("c")
    n_rays = dens_hbm.shape[0]
    per_w = n_rays // nw
    base = wid * per_w

    def chunk_body(ci, carry_unused):
        start = base + ci * c_rays
        pltpu.sync_copy(dens_hbm.at[pl.ds(start, c_rays), :], dens_v)
        pltpu.sync_copy(u_hbm.at[pl.ds(start, c_rays), :], u_v)

        @plsc.parallel_loop(0, c_rays, unroll=SC_UNROLL)
        def ray_body(r):
            run = jnp.float32(0.0)
            for j in range(n // 16):
                dv = dens_v[r, pl.ds(16 * j, 16)] + jnp.float32(1e-5)
                cj = plsc.cumsum(dv) + run
                cdf_v[pl.ds(r * n + 16 * j, 16)] = cj
                run = cj[15]
            rbase = jnp.full((16,), r * n - 1, jnp.int32)
            for j in range(n // 16):
                t = u_v[r, pl.ds(16 * j, 16)] * run
                lo = rbase
                for step in (64, 32, 16, 8, 4, 2, 1):
                    cg = plsc.load_gather(cdf_v, [lo + step])
                    lo = lo + jnp.where(cg <= t, step, 0)
                idx = jnp.minimum(lo - rbase, n - 1)
                zf = (idx.astype(jnp.float32)
                      * jnp.float32(DZ) + jnp.float32(MIN_D))
                z_v[r, pl.ds(16 * j, 16)] = zf
        pltpu.sync_copy(z_v, z_hbm.at[pl.ds(start, c_rays), :])
        return carry_unused

    lax.fori_loop(0, per_w // c_rays, chunk_body, 0)


def _sc_depths(dens, u):
    n_rays = dens.shape[0]
    n = N_SAMPLES
    mesh = plsc.VectorSubcoreMesh(core_axis_name="c", subcore_axis_name="s")
    return pl.kernel(
        _sc_body,
        out_type=jax.ShapeDtypeStruct((n_rays, n), jnp.float32),
        mesh=mesh,
        scratch_types=[
            pltpu.VMEM((SC_CHUNK, n), jnp.float32),
            pltpu.VMEM((SC_CHUNK, n), jnp.float32),
            pltpu.VMEM((SC_CHUNK * n,), jnp.float32),
            pltpu.VMEM((SC_CHUNK, n), jnp.float32),
        ],
        compiler_params=pltpu.CompilerParams(needs_layout_passes=False),
    )(dens, u)


def _tc_body(z_ref, org_ref, dir_ref, e3_ref, pts_ref):
    n = N_SAMPLES
    z = z_ref[...]
    o_b = jnp.dot(org_ref[...], e3_ref[...], preferred_element_type=jnp.float32)
    d_b = jnp.dot(dir_ref[...], e3_ref[...], preferred_element_type=jnp.float32)
    for k in range(3):
        pts_ref[k] = o_b[:, k * n:(k + 1) * n] + z * d_b[:, k * n:(k + 1) * n]


def _tc_points(z, origins, directions):
    n_rays = z.shape[0]
    n = N_SAMPLES
    r = BLOCK_R
    return pl.pallas_call(
        _tc_body,
        grid=(n_rays // r,),
        in_specs=[
            pl.BlockSpec((r, n), lambda i: (i, 0)),
            pl.BlockSpec((r, 3), lambda i: (i, 0)),
            pl.BlockSpec((r, 3), lambda i: (i, 0)),
            pl.BlockSpec((3, 3 * n), lambda i: (0, 0)),
        ],
        out_specs=pl.BlockSpec((3, r, n), lambda i: (0, i, 0)),
        out_shape=jax.ShapeDtypeStruct((3, n_rays, n), jnp.float32),
        compiler_params=pltpu.CompilerParams(
            dimension_semantics=("parallel",),
        ),
    )(z, origins, directions, _bcast_mat())


def kernel(origins, directions, density):
    n_rays = origins.shape[0]
    n = N_SAMPLES
    dens = jnp.squeeze(density, -1)
    u = _u_const(n_rays)
    z = _sc_depths(dens, u)
    pts = _tc_points(z, origins, directions)
    return pts.transpose(1, 2, 0), z.reshape(n_rays, n, 1)


# speedup vs baseline: 5.9122x; 5.9122x over previous
import functools

import jax
import jax.numpy as jnp
import numpy as np
from jax.experimental import pallas as pl
from jax.experimental.pallas import tpu as pltpu

N_SAMPLES = 128
MIN_D = 0.1
MAX_D = 6.0
DZ = (MAX_D - MIN_D) / (N_SAMPLES - 1)
BLOCK_R = 4096


def _threefry2x32_np(k0, k1, x0, x1):
    def rotl(x, d):
        return (x << np.uint32(d)) | (x >> np.uint32(32 - d))

    rots = ((13, 15, 26, 6), (17, 29, 16, 24))
    ks = (k0, k1, k0 ^ k1 ^ np.uint32(0x1BD11BDA))
    x0 = x0 + ks[0]
    x1 = x1 + ks[1]
    for i in range(5):
        for r in rots[i % 2]:
            x0 = x0 + x1
            x1 = rotl(x1, r)
            x1 = x0 ^ x1
        x0 = x0 + ks[(i + 1) % 3]
        x1 = x1 + ks[(i + 2) % 3] + np.uint32(i + 1)
    return x0, x1


@functools.cache
def _u_const(n_rays: int):
    size = n_rays * N_SAMPLES
    with np.errstate(over="ignore"):
        b0, b1 = _threefry2x32_np(
            np.uint32(0), np.uint32(42),
            np.zeros(size, np.uint32), np.arange(size, dtype=np.uint32),
        )
    bits = b0 ^ b1
    u = ((bits >> np.uint32(9)) | np.uint32(0x3F800000)).view(np.float32)
    u = np.maximum(np.float32(0.0), u - np.float32(1.0))
    return u.reshape(n_rays, N_SAMPLES)


@functools.cache
def _cumsum_mat():
    return jnp.asarray(np.triu(np.ones((N_SAMPLES, N_SAMPLES), np.float32)))


@functools.cache
def _bound_mat():
    n = N_SAMPLES
    m = np.zeros((n, 3 * n), np.float32)
    for b, lim in enumerate((64, 32, 96)):
        m[:lim, b * n:(b + 1) * n] = 1.0
    return jnp.asarray(m)


@functools.cache
def _bcast_mat():
    e = np.zeros((3, 3 * N_SAMPLES), np.float32)
    for k in range(3):
        e[k, k * N_SAMPLES:(k + 1) * N_SAMPLES] = 1.0
    return jnp.asarray(e)


def _body(dens_ref, u_ref, org_ref, dir_ref, ut_ref, e3_ref, mb_ref,
          pts_ref, len_ref):
    n = N_SAMPLES
    d = dens_ref[...] + 1e-5
    ones = jnp.ones((n, n), jnp.float32)
    s = jnp.dot(d, ones, preferred_element_type=jnp.float32)
    cum = jnp.dot(d, ut_ref[...], preferred_element_type=jnp.float32)
    t = u_ref[...] * s
    planes = jnp.dot(d, mb_ref[...], preferred_element_type=jnp.float32)
    b64 = planes[:, 0:n] <= t
    lo = jnp.where(b64, 64, 0)
    c32 = jnp.where(b64, planes[:, 2 * n:3 * n], planes[:, n:2 * n])
    lo = jnp.where(c32 <= t, lo + 32, lo)
    for step in (16, 8, 4, 2, 1):
        mid = lo + step
        c = jnp.take_along_axis(cum, mid - 1, axis=1, mode="promise_in_bounds")
        lo = jnp.where(c <= t, mid, lo)
    idx = jnp.minimum(lo, N_SAMPLES - 1).astype(jnp.float32)
    z = MIN_D + DZ * idx
    len_ref[...] = z
    o_b = jnp.dot(org_ref[...], e3_ref[...], preferred_element_type=jnp.float32)
    d_b = jnp.dot(dir_ref[...], e3_ref[...], preferred_element_type=jnp.float32)
    for k in range(3):
        pts_ref[k] = o_b[:, k * n:(k + 1) * n] + z * d_b[:, k * n:(k + 1) * n]


def kernel(origins, directions, density):
    n_rays = origins.shape[0]
    n = N_SAMPLES
    dens = jnp.squeeze(density, -1)
    u = _u_const(n_rays)
    ut = _cumsum_mat()
    grid = (n_rays // BLOCK_R,)
    r = BLOCK_R
    pts, lens = pl.pallas_call(
        _body,
        grid=grid,
        in_specs=[
            pl.BlockSpec((r, n), lambda i: (i, 0)),
            pl.BlockSpec((r, n), lambda i: (i, 0)),
            pl.BlockSpec((r, 3), lambda i: (i, 0)),
            pl.BlockSpec((r, 3), lambda i: (i, 0)),
            pl.BlockSpec((n, n), lambda i: (0, 0)),
            pl.BlockSpec((3, 3 * n), lambda i: (0, 0)),
            pl.BlockSpec((n, 3 * n), lambda i: (0, 0)),
        ],
        out_specs=[
            pl.BlockSpec((3, r, n), lambda i: (0, i, 0)),
            pl.BlockSpec((r, n), lambda i: (i, 0)),
        ],
        out_shape=[
            jax.ShapeDtypeStruct((3, n_rays, n), jnp.float32),
            jax.ShapeDtypeStruct((n_rays, n), jnp.float32),
        ],
        compiler_params=pltpu.CompilerParams(
            dimension_semantics=("parallel",),
        ),
    )(dens, u, origins, directions, ut, _bcast_mat(), _bound_mat())
    return pts.transpose(1, 2, 0), lens.reshape(n_rays, n, 1)
